# speedup vs baseline: 2.6584x; 2.6584x over previous
import functools

import jax
import jax.numpy as jnp
from jax import lax
from jax.experimental import pallas as pl
from jax.experimental.pallas import tpu as pltpu
from jax.experimental.pallas import tpu_sc as plsc

D_MODEL = 128
SCALE = float(D_MODEL) ** 0.5

NC = 2
NS = 16
NW = NC * NS

G = 128


def _scale_body(w_ref, o_ref):
    o_ref[...] = w_ref[...] * SCALE


def _scale_table(weights):
    n_vocab, d = weights.shape
    blk = 4000
    assert n_vocab % blk == 0
    return pl.pallas_call(
        _scale_body,
        out_shape=jax.ShapeDtypeStruct((n_vocab, d), jnp.float32),
        grid=(n_vocab // blk,),
        in_specs=[pl.BlockSpec((blk, d), lambda i: (i, 0))],
        out_specs=pl.BlockSpec((blk, d), lambda i: (i, 0)),
    )(weights)


def _make_gather(b_total):
    assert b_total % (NW * G) == 0
    rows_per_w = b_total // NW
    ng = rows_per_w // G

    mesh = plsc.VectorSubcoreMesh(core_axis_name="c", subcore_axis_name="s")

    @functools.partial(
        pl.kernel,
        out_type=jax.ShapeDtypeStruct((b_total, D_MODEL), jnp.float32),
        mesh=mesh,
        scratch_types=[
            pltpu.VMEM((ng, G), jnp.int32),
            pltpu.VMEM((G, D_MODEL), jnp.float32),
            pltpu.SemaphoreType.DMA,
        ],
    )
    def gather(table_hbm, idx_hbm, out_hbm, idx_v, rows_v, sem):
        wid = lax.axis_index("s") * NC + lax.axis_index("c")
        base = wid * rows_per_w
        pltpu.sync_copy(idx_hbm.at[wid], idx_v)

        def body(j, carry):
            pltpu.async_copy(table_hbm.at[idx_v.at[j]], rows_v, sem).wait()
            pltpu.sync_copy(rows_v, out_hbm.at[pl.ds(base + j * G, G)])
            return carry

        lax.fori_loop(0, ng, body, 0)

    return gather


def kernel(inputs, shared_weights):
    batch, hist = inputs.shape
    b_total = batch * hist
    idx = inputs.reshape(-1).astype(jnp.int32).reshape(NW, b_total // (NW * G), G)
    table = _scale_table(shared_weights)
    out = _make_gather(b_total)(table, idx)
    return out.reshape(batch, hist, D_MODEL)


# speedup vs baseline: 3.0020x; 1.1292x over previous
import functools

import jax
import jax.numpy as jnp
from jax import lax
from jax.experimental import pallas as pl
from jax.experimental.pallas import tpu as pltpu
from jax.experimental.pallas import tpu_sc as plsc

D_MODEL = 128
SCALE = float(D_MODEL) ** 0.5

NC = 2
NS = 16
NW = NC * NS

G = 128


def _scale_body(w_ref, o_ref):
    o_ref[...] = w_ref[...] * SCALE


def _scale_table(weights):
    n_vocab, d = weights.shape
    blk = 4000
    assert n_vocab % blk == 0
    return pl.pallas_call(
        _scale_body,
        out_shape=jax.ShapeDtypeStruct((n_vocab, d), jnp.float32),
        grid=(n_vocab // blk,),
        in_specs=[pl.BlockSpec((blk, d), lambda i: (i, 0))],
        out_specs=pl.BlockSpec((blk, d), lambda i: (i, 0)),
    )(weights)


NBUF = 4


def _make_gather(b_total):
    assert b_total % (NW * G) == 0
    rows_per_w = b_total // NW
    ng = rows_per_w // G
    assert ng % NBUF == 0 and ng >= 2 * NBUF

    mesh = plsc.VectorSubcoreMesh(core_axis_name="c", subcore_axis_name="s")

    @functools.partial(
        pl.kernel,
        out_type=jax.ShapeDtypeStruct((b_total, D_MODEL), jnp.float32),
        mesh=mesh,
        scratch_types=[
            pltpu.VMEM((ng, G), jnp.int32),
            *[pltpu.VMEM((G, D_MODEL), jnp.float32) for _ in range(NBUF)],
            *[pltpu.SemaphoreType.DMA for _ in range(2 * NBUF)],
        ],
    )
    def gather(table_hbm, idx_hbm, out_hbm, idx_v, *scratch):
        bufs = scratch[:NBUF]
        gs = scratch[NBUF:2 * NBUF]
        ss = scratch[2 * NBUF:]
        wid = lax.axis_index("s") * NC + lax.axis_index("c")
        base = wid * rows_per_w
        pltpu.sync_copy(idx_hbm.at[wid], idx_v)

        def g_start(j, b):
            pltpu.async_copy(table_hbm.at[idx_v.at[j]], bufs[b], gs[b])

        def g_wait(j, b):
            pltpu.make_async_copy(table_hbm.at[idx_v.at[j]], bufs[b], gs[b]).wait()

        def s_start(j, b):
            pltpu.async_copy(bufs[b], out_hbm.at[pl.ds(base + j * G, G)], ss[b])

        def s_wait(j, b):
            pltpu.make_async_copy(
                bufs[b], out_hbm.at[pl.ds(base + j * G, G)], ss[b]).wait()

        def step(j, b):
            g_wait(j, b)
            s_start(j, b)
            bg = (b + NBUF - 1) % NBUF
            s_wait(j - 1, bg)
            g_start(j + NBUF - 1, bg)

        for j in range(NBUF - 1):
            g_start(j, j)
        g_wait(0, 0)
        s_start(0, 0)
        g_start(NBUF - 1, NBUF - 1)
        for j in range(1, NBUF):
            step(j, j)

        def body(t, carry):
            j0 = t * NBUF
            for b in range(NBUF):
                step(j0 + b, b)
            return carry

        lax.fori_loop(1, ng // NBUF - 1, body, 0)

        j0 = ng - NBUF
        g_wait(j0, 0)
        s_start(j0, 0)
        s_wait(j0 - 1, NBUF - 1)
        g_start(ng - 1, NBUF - 1)
        for b in range(1, NBUF):
            g_wait(j0 + b, b)
            s_start(j0 + b, b)
        for b in range(NBUF):
            s_wait(j0 + b, b)

    return gather


def kernel(inputs, shared_weights):
    batch, hist = inputs.shape
    b_total = batch * hist
    idx = inputs.reshape(-1).astype(jnp.int32).reshape(NW, b_total // (NW * G), G)
    table = _scale_table(shared_weights)
    out = _make_gather(b_total)(table, idx)
    return out.reshape(batch, hist, D_MODEL)


# speedup vs baseline: 5.3851x; 1.7938x over previous
import functools

import jax
import jax.numpy as jnp
from jax import lax
from jax.experimental import pallas as pl
from jax.experimental.pallas import tpu as pltpu
from jax.experimental.pallas import tpu_sc as plsc

D_MODEL = 128
SCALE = float(D_MODEL) ** 0.5

NC = 2
NS = 16
NW = NC * NS

G = 128


def _scale_body(w_ref, o_ref):
    o_ref[...] = w_ref[...] * SCALE


def _scale_table(weights):
    n_vocab, d = weights.shape
    blk = 4000
    assert n_vocab % blk == 0
    return pl.pallas_call(
        _scale_body,
        out_shape=jax.ShapeDtypeStruct((n_vocab, d), jnp.float32),
        grid=(n_vocab // blk,),
        in_specs=[pl.BlockSpec((blk, d), lambda i: (i, 0))],
        out_specs=pl.BlockSpec((blk, d), lambda i: (i, 0)),
    )(weights)


NBUF = 4


def _make_gather(batch, hist):
    assert batch % NW == 0
    bat_per_w = batch // NW
    ng = bat_per_w
    assert ng % NBUF == 0 and ng >= 2 * NBUF

    mesh = plsc.VectorSubcoreMesh(core_axis_name="c", subcore_axis_name="s")

    @functools.partial(
        pl.kernel,
        out_type=jax.ShapeDtypeStruct((batch, hist, D_MODEL), jnp.float32),
        mesh=mesh,
        scratch_types=[
            pltpu.VMEM((ng, hist), jnp.int32),
            *[pltpu.VMEM((hist, D_MODEL), jnp.float32) for _ in range(NBUF)],
            *[pltpu.SemaphoreType.DMA for _ in range(2 * NBUF)],
        ],
    )
    def gather(table_hbm, idx_hbm, out_hbm, idx_v, *scratch):
        bufs = scratch[:NBUF]
        gs = scratch[NBUF:2 * NBUF]
        ss = scratch[2 * NBUF:]
        wid = lax.axis_index("s") * NC + lax.axis_index("c")
        base = wid * bat_per_w
        pltpu.sync_copy(idx_hbm.at[wid], idx_v)

        def g_start(j, b):
            pltpu.async_copy(table_hbm.at[idx_v.at[j]], bufs[b], gs[b])

        def g_wait(j, b):
            pltpu.make_async_copy(table_hbm.at[idx_v.at[j]], bufs[b], gs[b]).wait()

        def s_start(j, b):
            pltpu.async_copy(bufs[b], out_hbm.at[base + j], ss[b])

        def s_wait(j, b):
            pltpu.make_async_copy(bufs[b], out_hbm.at[base + j], ss[b]).wait()

        def step(j, b):
            g_wait(j, b)
            s_start(j, b)
            bg = (b + NBUF - 1) % NBUF
            s_wait(j - 1, bg)
            g_start(j + NBUF - 1, bg)

        for j in range(NBUF - 1):
            g_start(j, j)
        g_wait(0, 0)
        s_start(0, 0)
        g_start(NBUF - 1, NBUF - 1)
        for j in range(1, NBUF):
            step(j, j)

        def body(t, carry):
            j0 = t * NBUF
            for b in range(NBUF):
                step(j0 + b, b)
            return carry

        lax.fori_loop(1, ng // NBUF - 1, body, 0)

        j0 = ng - NBUF
        g_wait(j0, 0)
        s_start(j0, 0)
        s_wait(j0 - 1, NBUF - 1)
        g_start(ng - 1, NBUF - 1)
        for b in range(1, NBUF):
            g_wait(j0 + b, b)
            s_start(j0 + b, b)
        for b in range(NBUF):
            s_wait(j0 + b, b)

    return gather


def kernel(inputs, shared_weights):
    batch, hist = inputs.shape
    idx = inputs.astype(jnp.int32).reshape(NW, batch // NW, hist)
    table = _scale_table(shared_weights)
    return _make_gather(batch, hist)(table, idx)
